# baseline (speedup 1.0000x reference)
import functools

import jax
import jax.numpy as jnp
from jax import lax
from jax.experimental import pallas as pl
from jax.experimental.pallas import tpu as pltpu
from jax.experimental.pallas import tpu_sc as plsc

N_NODES = 10000
N_EDGES = 40000
NODE_IN = 32
EDGE_IN = 16
HIDDEN = 32
OUT = 32

NC = 2
NS = 16
NW = NC * NS
EP = 40960
BPW = EP // NW
CH = BPW // 128
NP = 10240
RPT = NP // NS

def _sc_mesh():
    return plsc.VectorSubcoreMesh(
        core_axis_name="c", subcore_axis_name="s",
        num_cores=NC, num_subcores=NS)


def _gather_body(table_hbm, idx_hbm, out_hbm, idx_v, rows_v, sem):
    wid = lax.axis_index("s") * NC + lax.axis_index("c")
    pltpu.sync_copy(idx_hbm.at[wid], idx_v)
    descs = [
        pltpu.async_copy(
            table_hbm.at[idx_v.at[j]], rows_v.at[pl.ds(j * 128, 128)], sem
        )
        for j in range(CH)
    ]
    for d in descs:
        d.wait()
    pltpu.sync_copy(rows_v, out_hbm.at[pl.ds(wid * BPW, BPW)])


def _make_sc_gather():
    return pl.kernel(
        _gather_body,
        out_type=jax.ShapeDtypeStruct((EP, 32), jnp.float32),
        mesh=_sc_mesh(),
        scratch_types=[
            pltpu.VMEM((CH, 128), jnp.int32),
            pltpu.VMEM((BPW, 32), jnp.float32),
            pltpu.SemaphoreType.DMA,
        ],
        compiler_params=pltpu.CompilerParams(use_tc_tiling_on_sc=False),
    )


def _scatter_body(with_counts, *refs):
    if with_counts:
        (msg_hbm, dst_hbm, z32_hbm, z16_hbm, ones_hbm,
         s_out, c_out, idx_v, msg_v, row_v, ones_v, cnt_v, sh_s, sh_c) = refs
    else:
        (msg_hbm, dst_hbm, z32_hbm,
         s_out, idx_v, msg_v, row_v, sh_s) = refs
    cid = lax.axis_index("c")
    sid = lax.axis_index("s")
    wid = sid * NC + cid
    pltpu.sync_copy(dst_hbm.at[wid], idx_v)
    pltpu.sync_copy(msg_hbm.at[pl.ds(wid * BPW, BPW)], msg_v)
    rb = sid * RPT
    pltpu.sync_copy(z32_hbm.at[pl.ds(rb, RPT)], row_v)
    pltpu.sync_copy(row_v, sh_s.at[pl.ds(rb, RPT)])
    if with_counts:
        pltpu.sync_copy(ones_hbm.at[:], ones_v)
        pltpu.sync_copy(z16_hbm.at[pl.ds(rb, RPT)], cnt_v)
        pltpu.sync_copy(cnt_v, sh_c.at[pl.ds(rb, RPT)])
    plsc.subcore_barrier()
    for j in range(CH):
        pltpu.sync_copy(
            msg_v.at[pl.ds(j * 128, 128)], sh_s.at[idx_v.at[j]], add=True
        )
    if with_counts:
        for j in range(CH):
            pltpu.sync_copy(
                ones_v.at[pl.ds(j * 128, 128)], sh_c.at[idx_v.at[j]], add=True
            )
    plsc.subcore_barrier()
    ob = cid * NP + rb
    pltpu.sync_copy(sh_s.at[pl.ds(rb, RPT)], row_v)
    pltpu.sync_copy(row_v, s_out.at[pl.ds(ob, RPT)])
    if with_counts:
        pltpu.sync_copy(sh_c.at[pl.ds(rb, RPT)], cnt_v)
        pltpu.sync_copy(cnt_v, c_out.at[pl.ds(ob, RPT)])


def _make_sc_scatter(with_counts):
    if with_counts:
        out_type = (
            jax.ShapeDtypeStruct((2 * NP, 32), jnp.float32),
            jax.ShapeDtypeStruct((2 * NP, 16), jnp.float32),
        )
        scratch = [
            pltpu.VMEM((CH, 128), jnp.int32),
            pltpu.VMEM((BPW, 32), jnp.float32),
            pltpu.VMEM((RPT, 32), jnp.float32),
            pltpu.VMEM((BPW, 16), jnp.float32),
            pltpu.VMEM((RPT, 16), jnp.float32),
            pltpu.VMEM_SHARED((NP, 32), jnp.float32),
            pltpu.VMEM_SHARED((NP, 16), jnp.float32),
        ]
    else:
        out_type = jax.ShapeDtypeStruct((2 * NP, 32), jnp.float32)
        scratch = [
            pltpu.VMEM((CH, 128), jnp.int32),
            pltpu.VMEM((BPW, 32), jnp.float32),
            pltpu.VMEM((RPT, 32), jnp.float32),
            pltpu.VMEM_SHARED((NP, 32), jnp.float32),
        ]
    return pl.kernel(
        functools.partial(_scatter_body, with_counts),
        out_type=out_type,
        mesh=_sc_mesh(),
        scratch_types=scratch,
        compiler_params=pltpu.CompilerParams(use_tc_tiling_on_sc=False),
    )


TE_MSG = 2000
TN = 2048


def _msg_body(ea_ref, xs_ref, w1_ref, b1_ref, rm_ref, tm_ref, w2_ref,
              b2r_ref, out_ref):
    ea = ea_ref[...]
    xs = xs_ref[...]
    he = jnp.maximum(
        jnp.dot(ea, w1_ref[...], preferred_element_type=jnp.float32)
        + b1_ref[...], 0.0)
    heb = he.astype(jnp.bfloat16)
    xsb = xs.astype(jnp.bfloat16)
    p = (jnp.dot(heb, rm_ref[...], preferred_element_type=jnp.float32)
         * jnp.dot(xsb, tm_ref[...], preferred_element_type=jnp.float32))
    out_ref[...] = (
        jnp.dot(p.astype(jnp.bfloat16), w2_ref[...],
                preferred_element_type=jnp.float32)
        + jnp.dot(xsb, b2r_ref[...], preferred_element_type=jnp.float32))


def _make_tc_msg(interpret=False):
    nb = -(-EP // TE_MSG)
    nreal = N_EDGES // TE_MSG
    return pl.pallas_call(
        _msg_body,
        grid=(nb,),
        in_specs=[
            pl.BlockSpec((TE_MSG, EDGE_IN),
                         lambda i: (jnp.minimum(i, nreal - 1), 0)),
            pl.BlockSpec((TE_MSG, 32),
                         lambda i: (jnp.minimum(i, nreal - 1), 0)),
            pl.BlockSpec((EDGE_IN, HIDDEN), lambda i: (0, 0)),
            pl.BlockSpec((1, HIDDEN), lambda i: (0, 0)),
            pl.BlockSpec((HIDDEN, HIDDEN * 32), lambda i: (0, 0)),
            pl.BlockSpec((32, HIDDEN * 32), lambda i: (0, 0)),
            pl.BlockSpec((HIDDEN * 32, 32), lambda i: (0, 0)),
            pl.BlockSpec((32, 32), lambda i: (0, 0)),
        ],
        out_specs=pl.BlockSpec((TE_MSG, 32), lambda i: (i, 0)),
        out_shape=jax.ShapeDtypeStruct((EP, 32), jnp.float32),
        interpret=interpret,
    )


def _upd_body(sp0_ref, sp1_ref, cp0_ref, cp1_ref, x_ref, r1_ref, bz1_ref,
              r2_ref, bz2_ref, h1_ref, xr2_ref):
    s = sp0_ref[...] + sp1_ref[...]
    cnt = cp0_ref[:, 0:1] + cp1_ref[:, 0:1]
    inv = 1.0 / jnp.maximum(cnt, 1.0)
    xr1 = (jnp.dot(x_ref[...], r1_ref[...], preferred_element_type=jnp.float32)
           + bz1_ref[...])
    h1 = jnp.maximum(s * inv + xr1, 0.0)
    h1_ref[...] = h1
    xr2_ref[...] = (
        jnp.dot(h1, r2_ref[...], preferred_element_type=jnp.float32)
        + bz2_ref[...])


def _make_tc_update(interpret=False):
    nb = NP // TN
    return pl.pallas_call(
        _upd_body,
        grid=(nb,),
        in_specs=[
            pl.BlockSpec((TN, 32), lambda i: (i, 0)),
            pl.BlockSpec((TN, 32), lambda i: (i + NP // TN, 0)),
            pl.BlockSpec((TN, 16), lambda i: (i, 0)),
            pl.BlockSpec((TN, 16), lambda i: (i + NP // TN, 0)),
            pl.BlockSpec((TN, 32), lambda i: (i, 0)),
            pl.BlockSpec((NODE_IN, HIDDEN), lambda i: (0, 0)),
            pl.BlockSpec((1, HIDDEN), lambda i: (0, 0)),
            pl.BlockSpec((HIDDEN, OUT), lambda i: (0, 0)),
            pl.BlockSpec((1, OUT), lambda i: (0, 0)),
        ],
        out_specs=[
            pl.BlockSpec((TN, 32), lambda i: (i, 0)),
            pl.BlockSpec((TN, OUT), lambda i: (i, 0)),
        ],
        out_shape=[
            jax.ShapeDtypeStruct((NP, HIDDEN), jnp.float32),
            jax.ShapeDtypeStruct((NP, OUT), jnp.float32),
        ],
        interpret=interpret,
    )


def _fin_body(sp0_ref, sp1_ref, cp0_ref, cp1_ref, xr2_ref, out_ref):
    i = pl.program_id(0)
    nb = pl.num_programs(0)
    s = sp0_ref[...] + sp1_ref[...]
    cnt = cp0_ref[:, 0:1] + cp1_ref[:, 0:1]
    inv = 1.0 / jnp.maximum(cnt, 1.0)
    h2 = jnp.maximum(s * inv + xr2_ref[...], 0.0)
    rows = lax.broadcasted_iota(jnp.int32, (TN, OUT), 0) + i * TN
    contrib = jnp.sum(
        jnp.where(rows < N_NODES, h2, 0.0), axis=0, keepdims=True)

    @pl.when(i == 0)
    def _():
        out_ref[...] = jnp.zeros((1, OUT), jnp.float32)

    out_ref[...] += contrib

    @pl.when(i == nb - 1)
    def _():
        out_ref[...] = out_ref[...] * (1.0 / N_NODES)


def _make_tc_final(interpret=False):
    nb = NP // TN
    return pl.pallas_call(
        _fin_body,
        grid=(nb,),
        in_specs=[
            pl.BlockSpec((TN, 32), lambda i: (i, 0)),
            pl.BlockSpec((TN, 32), lambda i: (i + NP // TN, 0)),
            pl.BlockSpec((TN, 16), lambda i: (i, 0)),
            pl.BlockSpec((TN, 16), lambda i: (i + NP // TN, 0)),
            pl.BlockSpec((TN, OUT), lambda i: (i, 0)),
        ],
        out_specs=pl.BlockSpec((1, OUT), lambda i: (0, 0)),
        out_shape=jax.ShapeDtypeStruct((1, OUT), jnp.float32),
        interpret=interpret,
    )


def _rmat():
    return jnp.repeat(jnp.eye(HIDDEN, dtype=jnp.float32), 32, axis=1)


def _tmat():
    return jnp.tile(jnp.eye(32, dtype=jnp.float32), (1, HIDDEN))


def kernel(x, edge_index, edge_attr, en1_w1, en1_b1, en1_w2, en1_b2,
           root1, bias1, en2_w1, en2_b1, en2_w2, en2_b2, root2, bias2):
    src = edge_index[0]
    dst = edge_index[1]
    pad_e = EP - N_EDGES
    src_i = jnp.concatenate([src, jnp.zeros((pad_e,), jnp.int32)]
                            ).reshape(NW, CH, 128)
    dst_i = jnp.concatenate([dst, jnp.full((pad_e,), N_NODES, jnp.int32)]
                            ).reshape(NW, CH, 128)
    x_p = jnp.concatenate([x, jnp.zeros((NP - N_NODES, NODE_IN), jnp.float32)])
    z32 = jnp.zeros((NP, 32), jnp.float32)
    z16 = jnp.zeros((NP, 16), jnp.float32)
    ones16 = jnp.ones((BPW, 16), jnp.float32)
    rm = _rmat().astype(jnp.bfloat16)
    tm = _tmat().astype(jnp.bfloat16)
    w2f1 = en1_w2.reshape(HIDDEN * NODE_IN, HIDDEN).astype(jnp.bfloat16)
    w2f2 = en2_w2.reshape(HIDDEN * HIDDEN, OUT).astype(jnp.bfloat16)
    b2r1 = en1_b2.reshape(NODE_IN, HIDDEN).astype(jnp.bfloat16)
    b2r2 = en2_b2.reshape(HIDDEN, OUT).astype(jnp.bfloat16)

    sc_gather = _make_sc_gather()
    tc_msg = _make_tc_msg()
    sc_scatter_c = _make_sc_scatter(True)
    sc_scatter = _make_sc_scatter(False)
    tc_update = _make_tc_update()
    tc_final = _make_tc_final()

    xs1 = sc_gather(x_p, src_i)
    msg1 = tc_msg(edge_attr, xs1, en1_w1, en1_b1.reshape(1, HIDDEN),
                  rm, tm, w2f1, b2r1)
    s1, c1 = sc_scatter_c(msg1, dst_i, z32, z16, ones16)
    h1, xr2 = tc_update(s1, s1, c1, c1, x_p, root1, bias1.reshape(1, HIDDEN),
                        root2, bias2.reshape(1, OUT))
    xs2 = sc_gather(h1, src_i)
    msg2 = tc_msg(edge_attr, xs2, en2_w1, en2_b1.reshape(1, HIDDEN),
                  rm, tm, w2f2, b2r2)
    s2 = sc_scatter(msg2, dst_i, z32)
    return tc_final(s2, s2, c1, c1, xr2)


# speedup vs baseline: 1.1353x; 1.1353x over previous
import functools

import jax
import jax.numpy as jnp
from jax import lax
from jax.experimental import pallas as pl
from jax.experimental.pallas import tpu as pltpu
from jax.experimental.pallas import tpu_sc as plsc

N_NODES = 10000
N_EDGES = 40000
NODE_IN = 32
EDGE_IN = 16
HIDDEN = 32
OUT = 32

NC = 2
NS = 16
NW = NC * NS
EP = 40960
BPW = EP // NW
CH = BPW // 128
NP = 10240
RPT = NP // NS

def _sc_mesh():
    return plsc.VectorSubcoreMesh(
        core_axis_name="c", subcore_axis_name="s",
        num_cores=NC, num_subcores=NS)


def _gather_body(table_hbm, idx_hbm, out_hbm, idx_v, rows_v, sem):
    wid = lax.axis_index("s") * NC + lax.axis_index("c")
    pltpu.sync_copy(idx_hbm.at[wid], idx_v)
    descs = [
        pltpu.async_copy(
            table_hbm.at[idx_v.at[j]], rows_v.at[pl.ds(j * 128, 128)], sem
        )
        for j in range(CH)
    ]
    for d in descs:
        d.wait()
    pltpu.sync_copy(rows_v, out_hbm.at[pl.ds(wid * BPW, BPW)])


def _make_sc_gather():
    return pl.kernel(
        _gather_body,
        out_type=jax.ShapeDtypeStruct((EP, 32), jnp.float32),
        mesh=_sc_mesh(),
        scratch_types=[
            pltpu.VMEM((CH, 128), jnp.int32),
            pltpu.VMEM((BPW, 32), jnp.float32),
            pltpu.SemaphoreType.DMA,
        ],
        compiler_params=pltpu.CompilerParams(use_tc_tiling_on_sc=False),
    )


def _scatter_body(with_counts, *refs):
    if with_counts:
        (msg_hbm, dst_hbm, z32_hbm, z16_hbm, ones_hbm,
         s_out, c_out, idx_v, msg_v, row_v, ones_v, cnt_v, sh_s, sh_c) = refs
    else:
        (msg_hbm, dst_hbm, z32_hbm,
         s_out, idx_v, msg_v, row_v, sh_s) = refs
    cid = lax.axis_index("c")
    sid = lax.axis_index("s")
    wid = sid * NC + cid
    pltpu.sync_copy(dst_hbm.at[wid], idx_v)
    pltpu.sync_copy(msg_hbm.at[pl.ds(wid * BPW, BPW)], msg_v)
    rb = sid * RPT
    pltpu.sync_copy(z32_hbm.at[pl.ds(rb, RPT)], row_v)
    pltpu.sync_copy(row_v, sh_s.at[pl.ds(rb, RPT)])
    if with_counts:
        pltpu.sync_copy(ones_hbm.at[:], ones_v)
        pltpu.sync_copy(z16_hbm.at[pl.ds(rb, RPT)], cnt_v)
        pltpu.sync_copy(cnt_v, sh_c.at[pl.ds(rb, RPT)])
    plsc.subcore_barrier()
    for j in range(CH):
        pltpu.sync_copy(
            msg_v.at[pl.ds(j * 128, 128)], sh_s.at[idx_v.at[j]], add=True
        )
    if with_counts:
        for j in range(CH):
            pltpu.sync_copy(
                ones_v.at[pl.ds(j * 128, 128)], sh_c.at[idx_v.at[j]], add=True
            )
    plsc.subcore_barrier()
    ob = cid * NP + rb
    pltpu.sync_copy(sh_s.at[pl.ds(rb, RPT)], row_v)
    pltpu.sync_copy(row_v, s_out.at[pl.ds(ob, RPT)])
    if with_counts:
        pltpu.sync_copy(sh_c.at[pl.ds(rb, RPT)], cnt_v)
        pltpu.sync_copy(cnt_v, c_out.at[pl.ds(ob, RPT)])


def _make_sc_scatter(with_counts):
    if with_counts:
        out_type = (
            jax.ShapeDtypeStruct((2 * NP, 32), jnp.float32),
            jax.ShapeDtypeStruct((2 * NP, 16), jnp.float32),
        )
        scratch = [
            pltpu.VMEM((CH, 128), jnp.int32),
            pltpu.VMEM((BPW, 32), jnp.float32),
            pltpu.VMEM((RPT, 32), jnp.float32),
            pltpu.VMEM((BPW, 16), jnp.float32),
            pltpu.VMEM((RPT, 16), jnp.float32),
            pltpu.VMEM_SHARED((NP, 32), jnp.float32),
            pltpu.VMEM_SHARED((NP, 16), jnp.float32),
        ]
    else:
        out_type = jax.ShapeDtypeStruct((2 * NP, 32), jnp.float32)
        scratch = [
            pltpu.VMEM((CH, 128), jnp.int32),
            pltpu.VMEM((BPW, 32), jnp.float32),
            pltpu.VMEM((RPT, 32), jnp.float32),
            pltpu.VMEM_SHARED((NP, 32), jnp.float32),
        ]
    return pl.kernel(
        functools.partial(_scatter_body, with_counts),
        out_type=out_type,
        mesh=_sc_mesh(),
        scratch_types=scratch,
        compiler_params=pltpu.CompilerParams(use_tc_tiling_on_sc=False),
    )


TE_MSG = 2000
TN = 2048


def _msg_body(ea_ref, xs_ref, w1r_ref, b1r_ref, tm_ref, w2_ref,
              b2r_ref, out_ref):
    ea = ea_ref[...]
    xs = xs_ref[...]
    p = (jnp.maximum(
            jnp.dot(ea, w1r_ref[...], preferred_element_type=jnp.float32)
            + b1r_ref[...], 0.0)
         * jnp.dot(xs, tm_ref[...], preferred_element_type=jnp.float32))
    out_ref[...] = (
        jnp.dot(p, w2_ref[...], preferred_element_type=jnp.float32)
        + jnp.dot(xs, b2r_ref[...], preferred_element_type=jnp.float32))


def _make_tc_msg(interpret=False):
    nb = -(-EP // TE_MSG)
    nreal = N_EDGES // TE_MSG
    return pl.pallas_call(
        _msg_body,
        grid=(nb,),
        in_specs=[
            pl.BlockSpec((TE_MSG, EDGE_IN),
                         lambda i: (jnp.minimum(i, nreal - 1), 0)),
            pl.BlockSpec((TE_MSG, 32),
                         lambda i: (jnp.minimum(i, nreal - 1), 0)),
            pl.BlockSpec((EDGE_IN, HIDDEN * 32), lambda i: (0, 0)),
            pl.BlockSpec((1, HIDDEN * 32), lambda i: (0, 0)),
            pl.BlockSpec((32, HIDDEN * 32), lambda i: (0, 0)),
            pl.BlockSpec((HIDDEN * 32, 32), lambda i: (0, 0)),
            pl.BlockSpec((32, 32), lambda i: (0, 0)),
        ],
        out_specs=pl.BlockSpec((TE_MSG, 32), lambda i: (i, 0)),
        out_shape=jax.ShapeDtypeStruct((EP, 32), jnp.float32),
        interpret=interpret,
    )


def _upd_body(sp0_ref, sp1_ref, cp0_ref, cp1_ref, x_ref, r1_ref, bz1_ref,
              r2_ref, bz2_ref, h1_ref, xr2_ref):
    s = sp0_ref[...] + sp1_ref[...]
    cnt = cp0_ref[:, 0:1] + cp1_ref[:, 0:1]
    inv = 1.0 / jnp.maximum(cnt, 1.0)
    xr1 = (jnp.dot(x_ref[...], r1_ref[...], preferred_element_type=jnp.float32)
           + bz1_ref[...])
    h1 = jnp.maximum(s * inv + xr1, 0.0)
    h1_ref[...] = h1
    xr2_ref[...] = (
        jnp.dot(h1, r2_ref[...], preferred_element_type=jnp.float32)
        + bz2_ref[...])


def _make_tc_update(interpret=False):
    nb = NP // TN
    return pl.pallas_call(
        _upd_body,
        grid=(nb,),
        in_specs=[
            pl.BlockSpec((TN, 32), lambda i: (i, 0)),
            pl.BlockSpec((TN, 32), lambda i: (i + NP // TN, 0)),
            pl.BlockSpec((TN, 16), lambda i: (i, 0)),
            pl.BlockSpec((TN, 16), lambda i: (i + NP // TN, 0)),
            pl.BlockSpec((TN, 32), lambda i: (i, 0)),
            pl.BlockSpec((NODE_IN, HIDDEN), lambda i: (0, 0)),
            pl.BlockSpec((1, HIDDEN), lambda i: (0, 0)),
            pl.BlockSpec((HIDDEN, OUT), lambda i: (0, 0)),
            pl.BlockSpec((1, OUT), lambda i: (0, 0)),
        ],
        out_specs=[
            pl.BlockSpec((TN, 32), lambda i: (i, 0)),
            pl.BlockSpec((TN, OUT), lambda i: (i, 0)),
        ],
        out_shape=[
            jax.ShapeDtypeStruct((NP, HIDDEN), jnp.float32),
            jax.ShapeDtypeStruct((NP, OUT), jnp.float32),
        ],
        interpret=interpret,
    )


def _fin_body(sp0_ref, sp1_ref, cp0_ref, cp1_ref, xr2_ref, out_ref):
    i = pl.program_id(0)
    nb = pl.num_programs(0)
    s = sp0_ref[...] + sp1_ref[...]
    cnt = cp0_ref[:, 0:1] + cp1_ref[:, 0:1]
    inv = 1.0 / jnp.maximum(cnt, 1.0)
    h2 = jnp.maximum(s * inv + xr2_ref[...], 0.0)
    rows = lax.broadcasted_iota(jnp.int32, (TN, OUT), 0) + i * TN
    contrib = jnp.sum(
        jnp.where(rows < N_NODES, h2, 0.0), axis=0, keepdims=True)

    @pl.when(i == 0)
    def _():
        out_ref[...] = jnp.zeros((1, OUT), jnp.float32)

    out_ref[...] += contrib

    @pl.when(i == nb - 1)
    def _():
        out_ref[...] = out_ref[...] * (1.0 / N_NODES)


def _make_tc_final(interpret=False):
    nb = NP // TN
    return pl.pallas_call(
        _fin_body,
        grid=(nb,),
        in_specs=[
            pl.BlockSpec((TN, 32), lambda i: (i, 0)),
            pl.BlockSpec((TN, 32), lambda i: (i + NP // TN, 0)),
            pl.BlockSpec((TN, 16), lambda i: (i, 0)),
            pl.BlockSpec((TN, 16), lambda i: (i + NP // TN, 0)),
            pl.BlockSpec((TN, OUT), lambda i: (i, 0)),
        ],
        out_specs=pl.BlockSpec((1, OUT), lambda i: (0, 0)),
        out_shape=jax.ShapeDtypeStruct((1, OUT), jnp.float32),
        interpret=interpret,
    )


def _rmat():
    return jnp.repeat(jnp.eye(HIDDEN, dtype=jnp.float32), 32, axis=1)


def _tmat():
    return jnp.tile(jnp.eye(32, dtype=jnp.float32), (1, HIDDEN))


def kernel(x, edge_index, edge_attr, en1_w1, en1_b1, en1_w2, en1_b2,
           root1, bias1, en2_w1, en2_b1, en2_w2, en2_b2, root2, bias2):
    src = edge_index[0]
    dst = edge_index[1]
    pad_e = EP - N_EDGES
    src_i = jnp.concatenate([src, jnp.zeros((pad_e,), jnp.int32)]
                            ).reshape(NW, CH, 128)
    dst_i = jnp.concatenate([dst, jnp.full((pad_e,), N_NODES, jnp.int32)]
                            ).reshape(NW, CH, 128)
    x_p = jnp.concatenate([x, jnp.zeros((NP - N_NODES, NODE_IN), jnp.float32)])
    z32 = jnp.zeros((NP, 32), jnp.float32)
    z16 = jnp.zeros((NP, 16), jnp.float32)
    ones16 = jnp.ones((BPW, 16), jnp.float32)
    rm = _rmat()
    tm = _tmat()
    w1r1 = en1_w1 @ rm
    b1r1 = (en1_b1 @ rm).reshape(1, HIDDEN * 32)
    w1r2 = en2_w1 @ rm
    b1r2 = (en2_b1 @ rm).reshape(1, HIDDEN * 32)
    w2f1 = en1_w2.reshape(HIDDEN * NODE_IN, HIDDEN)
    w2f2 = en2_w2.reshape(HIDDEN * HIDDEN, OUT)
    b2r1 = en1_b2.reshape(NODE_IN, HIDDEN)
    b2r2 = en2_b2.reshape(HIDDEN, OUT)

    sc_gather = _make_sc_gather()
    tc_msg = _make_tc_msg()
    sc_scatter_c = _make_sc_scatter(True)
    sc_scatter = _make_sc_scatter(False)
    tc_update = _make_tc_update()
    tc_final = _make_tc_final()

    xs1 = sc_gather(x_p, src_i)
    msg1 = tc_msg(edge_attr, xs1, w1r1, b1r1, tm, w2f1, b2r1)
    s1, c1 = sc_scatter_c(msg1, dst_i, z32, z16, ones16)
    h1, xr2 = tc_update(s1, s1, c1, c1, x_p, root1, bias1.reshape(1, HIDDEN),
                        root2, bias2.reshape(1, OUT))
    xs2 = sc_gather(h1, src_i)
    msg2 = tc_msg(edge_attr, xs2, w1r2, b1r2, tm, w2f2, b2r2)
    s2 = sc_scatter(msg2, dst_i, z32)
    return tc_final(s2, s2, c1, c1, xr2)
